# baseline (speedup 1.0000x reference)
import functools

import jax
import jax.numpy as jnp
from jax import lax
from jax.experimental import pallas as pl
from jax.experimental.pallas import tpu as pltpu
from jax.experimental.pallas import tpu_sc as plsc

_D = 32
_C = 1024
_BB = 64


def _sc_gather(idx_flat, token_table, N, lo, hi):
    NC, NS = 2, 16
    NW = NC * NS
    per_w = (hi - lo) // NW
    n_chunks = per_w // _C

    mesh = plsc.VectorSubcoreMesh(core_axis_name="c", subcore_axis_name="s")

    @functools.partial(
        pl.kernel,
        mesh=mesh,
        out_type=jax.ShapeDtypeStruct((N, _D), jnp.float32),
        scratch_types=[
            pltpu.VMEM((2, _C), jnp.int32),
            pltpu.VMEM((2, _C, _D), jnp.float32),
            pltpu.SemaphoreType.DMA,
            pltpu.SemaphoreType.DMA,
            pltpu.SemaphoreType.DMA,
            pltpu.SemaphoreType.DMA,
        ],
        compiler_params=pltpu.CompilerParams(use_tc_tiling_on_sc=False),
    )
    def k(idx_hbm, table_hbm, out_hbm, idx_v, rows_v, gsem0, gsem1, osem0, osem1):
        wid = lax.axis_index("s") * NC + lax.axis_index("c")
        base = lo + wid * per_w
        gsems = (gsem0, gsem1)
        osems = (osem0, osem1)

        pltpu.sync_copy(idx_hbm.at[pl.ds(base, _C)], idx_v.at[0])
        pltpu.async_copy(table_hbm.at[idx_v.at[0]], rows_v.at[0], gsem0)

        def chunk_pair(g2, carry):
            for s in (0, 1):
                g = 2 * g2 + s
                ns = 1 - s
                off = base + g * _C

                @pl.when((g + 1 < n_chunks) & (g >= 1))
                def _():
                    pltpu.make_async_copy(
                        rows_v.at[ns],
                        out_hbm.at[pl.ds(off - _C, _C)],
                        osems[ns],
                    ).wait()

                @pl.when(g + 1 < n_chunks)
                def _():
                    pltpu.sync_copy(
                        idx_hbm.at[pl.ds(off + _C, _C)], idx_v.at[ns]
                    )
                    pltpu.async_copy(
                        table_hbm.at[idx_v.at[ns]], rows_v.at[ns], gsems[ns]
                    )

                pltpu.make_async_copy(
                    table_hbm.at[idx_v.at[s]], rows_v.at[s], gsems[s]
                ).wait()
                pltpu.async_copy(
                    rows_v.at[s], out_hbm.at[pl.ds(off, _C)], osems[s]
                )
            return carry

        lax.fori_loop(0, n_chunks // 2, chunk_pair, 0)
        for g in (n_chunks - 2, n_chunks - 1):
            pltpu.make_async_copy(
                rows_v.at[g % 2],
                out_hbm.at[pl.ds(base + g * _C, _C)],
                osems[g % 2],
            ).wait()

    return k(idx_flat, token_table)


def _tc_body(x_ref, p_ref, *rest):
    o_ref = rest[-1]
    x = (x_ref[...] + p_ref[...][None]).astype(jnp.bfloat16)
    parts = [
        lax.slice(x, (0, 0, k * _D), (_BB, 2 * _D, (k + 1) * _D))
        for k in range(4)
    ]
    u = lax.broadcasted_iota(jnp.int32, (2 * _D, 256), 0)
    tt = lax.broadcasted_iota(jnp.int32, (2 * _D, 256), 1)
    acc = None
    for k in range(4):
        w = (u * 4 + k == tt).astype(jnp.bfloat16)
        y = lax.dot_general(
            parts[k], w, (((1,), (0,)), ((), ())),
            preferred_element_type=jnp.float32,
        )
        acc = y if acc is None else acc + y
    o_ref[...] = acc


_P = 2


def kernel(idx, token_table, pos_table):
    B, T = idx.shape
    N = B * T
    pblk = B // _P // _BB
    idx_flat = idx.reshape(-1)
    pos64 = lax.slice(token_table, (0, 0), (T, _D)).reshape(2 * _D, 128)

    rows64 = [
        _sc_gather(
            idx_flat, token_table, N, p * N // _P, (p + 1) * N // _P
        ).reshape(B, 2 * _D, 128)
        for p in range(_P)
    ]

    out = None
    for p in range(_P):
        def in_map(i, p=p):
            return (i + p * pblk, 0, 0)

        in_specs = [
            pl.BlockSpec((_BB, 2 * _D, 128), in_map),
            pl.BlockSpec((2 * _D, 128), lambda i: (0, 0)),
        ]
        operands = [rows64[p], pos64]
        aliases = {}
        if out is not None:
            in_specs.append(pl.BlockSpec((_BB, _D, T), lambda i: (0, 0, 0)))
            operands.append(out)
            aliases = {2: 0}
        out = pl.pallas_call(
            _tc_body,
            grid=(pblk,),
            in_specs=in_specs,
            out_specs=pl.BlockSpec((_BB, _D, T), in_map),
            out_shape=jax.ShapeDtypeStruct((B, _D, T), jnp.float32),
            input_output_aliases=aliases,
        )(*operands)
    return out.transpose(0, 2, 1)


# speedup vs baseline: 1.0322x; 1.0322x over previous
import functools

import jax
import jax.numpy as jnp
from jax import lax
from jax.experimental import pallas as pl
from jax.experimental.pallas import tpu as pltpu
from jax.experimental.pallas import tpu_sc as plsc

_D = 32
_C = 1024
_BB = 64


def _sc_gather(idx_flat, token_table, N, lo, hi):
    NC, NS = 2, 16
    NW = NC * NS
    per_w = (hi - lo) // NW
    n_chunks = per_w // _C

    mesh = plsc.VectorSubcoreMesh(core_axis_name="c", subcore_axis_name="s")

    @functools.partial(
        pl.kernel,
        mesh=mesh,
        out_type=jax.ShapeDtypeStruct((N, _D), jnp.float32),
        scratch_types=[
            pltpu.VMEM((2, _C), jnp.int32),
            pltpu.VMEM((2, _C, _D), jnp.float32),
            pltpu.SemaphoreType.DMA,
            pltpu.SemaphoreType.DMA,
            pltpu.SemaphoreType.DMA,
            pltpu.SemaphoreType.DMA,
        ],
        compiler_params=pltpu.CompilerParams(use_tc_tiling_on_sc=False),
    )
    def k(idx_hbm, table_hbm, out_hbm, idx_v, rows_v, gsem0, gsem1, osem0, osem1):
        wid = lax.axis_index("s") * NC + lax.axis_index("c")
        base = lo + wid * per_w
        gsems = (gsem0, gsem1)
        osems = (osem0, osem1)

        def store_chunk(s, off, wait):
            gc = off // _C
            o0 = (gc // 2) * (8 * 256) + (gc % 2) * 128
            for r8 in range(8):
                args = (
                    rows_v.at[s].at[pl.ds(r8 * 128, 128)],
                    out_hbm.at[pl.ds(o0 + r8 * 256, 128)],
                    osems[s],
                )
                if wait:
                    pltpu.make_async_copy(*args).wait()
                else:
                    pltpu.async_copy(*args)

        pltpu.sync_copy(idx_hbm.at[pl.ds(base, _C)], idx_v.at[0])
        pltpu.async_copy(table_hbm.at[idx_v.at[0]], rows_v.at[0], gsem0)

        def chunk_pair(g2, carry):
            for s in (0, 1):
                g = 2 * g2 + s
                ns = 1 - s
                off = base + g * _C

                @pl.when((g + 1 < n_chunks) & (g >= 1))
                def _():
                    store_chunk(ns, off - _C, wait=True)

                @pl.when(g + 1 < n_chunks)
                def _():
                    pltpu.sync_copy(
                        idx_hbm.at[pl.ds(off + _C, _C)], idx_v.at[ns]
                    )
                    pltpu.async_copy(
                        table_hbm.at[idx_v.at[ns]], rows_v.at[ns], gsems[ns]
                    )

                pltpu.make_async_copy(
                    table_hbm.at[idx_v.at[s]], rows_v.at[s], gsems[s]
                ).wait()
                store_chunk(s, off, wait=False)
            return carry

        lax.fori_loop(0, n_chunks // 2, chunk_pair, 0)
        for g in (n_chunks - 2, n_chunks - 1):
            store_chunk(g % 2, base + g * _C, wait=True)

    return k(idx_flat, token_table)


def _tc_body(x_ref, p_ref, *rest):
    o_ref = rest[-1]
    x = (x_ref[...] + p_ref[...][None]).astype(jnp.bfloat16)
    parts = [
        lax.slice(x, (0, 0, k * _D), (_BB, 2 * _D, (k + 1) * _D))
        for k in range(4)
    ]
    u = lax.broadcasted_iota(jnp.int32, (2 * _D, 256), 0)
    tt = lax.broadcasted_iota(jnp.int32, (2 * _D, 256), 1)
    acc = None
    for k in range(4):
        w = (u * 4 + k == tt).astype(jnp.bfloat16)
        y = lax.dot_general(
            parts[k], w, (((1,), (0,)), ((), ())),
            preferred_element_type=jnp.float32,
        )
        acc = y if acc is None else acc + y
    o_ref[...] = acc


_P = 4


def kernel(idx, token_table, pos_table):
    B, T = idx.shape
    N = B * T
    pblk = B // _P // _BB
    idx_flat = (
        idx.reshape(B // 8, 8, T // 128, 128).transpose(0, 2, 1, 3).reshape(-1)
    )
    pos64 = lax.slice(token_table, (0, 0), (T, _D)).reshape(2 * _D, 128)

    rows64 = [
        _sc_gather(
            idx_flat, token_table, N, p * N // _P, (p + 1) * N // _P
        ).reshape(B, 2 * _D, 128)
        for p in range(_P)
    ]

    out = None
    for p in range(_P):
        def in_map(i, p=p):
            return (i + p * pblk, 0, 0)

        in_specs = [
            pl.BlockSpec((_BB, 2 * _D, 128), in_map),
            pl.BlockSpec((2 * _D, 128), lambda i: (0, 0)),
        ]
        operands = [rows64[p], pos64]
        aliases = {}
        if out is not None:
            in_specs.append(pl.BlockSpec((_BB, _D, T), lambda i: (0, 0, 0)))
            operands.append(out)
            aliases = {2: 0}
        out = pl.pallas_call(
            _tc_body,
            grid=(pblk,),
            in_specs=in_specs,
            out_specs=pl.BlockSpec((_BB, _D, T), in_map),
            out_shape=jax.ShapeDtypeStruct((B, _D, T), jnp.float32),
            input_output_aliases=aliases,
        )(*operands)
    return out.transpose(0, 2, 1)


# speedup vs baseline: 1.0410x; 1.0085x over previous
import functools

import jax
import jax.numpy as jnp
from jax import lax
from jax.experimental import pallas as pl
from jax.experimental.pallas import tpu as pltpu
from jax.experimental.pallas import tpu_sc as plsc

_D = 32
_C = 1024
_BB = 64


def _sc_gather(idx_flat, token_table, N, lo, hi):
    NC, NS = 2, 16
    NW = NC * NS
    per_w = (hi - lo) // NW
    n_chunks = per_w // _C

    mesh = plsc.VectorSubcoreMesh(core_axis_name="c", subcore_axis_name="s")

    @functools.partial(
        pl.kernel,
        mesh=mesh,
        out_type=jax.ShapeDtypeStruct((N, _D), jnp.float32),
        scratch_types=[
            pltpu.VMEM((2, _C), jnp.int32),
            pltpu.VMEM((2, _C, _D), jnp.float32),
            pltpu.SemaphoreType.DMA,
            pltpu.SemaphoreType.DMA,
            pltpu.SemaphoreType.DMA,
            pltpu.SemaphoreType.DMA,
        ],
        compiler_params=pltpu.CompilerParams(use_tc_tiling_on_sc=False),
    )
    def k(idx_hbm, table_hbm, out_hbm, idx_v, rows_v, gsem0, gsem1, osem0, osem1):
        wid = lax.axis_index("s") * NC + lax.axis_index("c")
        base = lo + wid * per_w
        gsems = (gsem0, gsem1)
        osems = (osem0, osem1)

        def store_chunk(s, off, wait):
            gc = off // _C
            o0 = (gc // 2) * (8 * 256) + (gc % 2) * 128
            for r8 in range(8):
                args = (
                    rows_v.at[s].at[pl.ds(r8 * 128, 128)],
                    out_hbm.at[pl.ds(o0 + r8 * 256, 128)],
                    osems[s],
                )
                if wait:
                    pltpu.make_async_copy(*args).wait()
                else:
                    pltpu.async_copy(*args)

        pltpu.sync_copy(idx_hbm.at[pl.ds(base, _C)], idx_v.at[0])
        pltpu.async_copy(table_hbm.at[idx_v.at[0]], rows_v.at[0], gsem0)

        def chunk_pair(g2, carry):
            for s in (0, 1):
                g = 2 * g2 + s
                ns = 1 - s
                off = base + g * _C

                @pl.when((g + 1 < n_chunks) & (g >= 1))
                def _():
                    store_chunk(ns, off - _C, wait=True)

                @pl.when(g + 1 < n_chunks)
                def _():
                    pltpu.sync_copy(
                        idx_hbm.at[pl.ds(off + _C, _C)], idx_v.at[ns]
                    )
                    pltpu.async_copy(
                        table_hbm.at[idx_v.at[ns]], rows_v.at[ns], gsems[ns]
                    )

                pltpu.make_async_copy(
                    table_hbm.at[idx_v.at[s]], rows_v.at[s], gsems[s]
                ).wait()
                store_chunk(s, off, wait=False)
            return carry

        lax.fori_loop(0, n_chunks // 2, chunk_pair, 0)
        for g in (n_chunks - 2, n_chunks - 1):
            store_chunk(g % 2, base + g * _C, wait=True)

    return k(idx_flat, token_table)


def _tc_body(x_ref, p_ref, *rest):
    o_ref = rest[-1]
    x = (x_ref[...] + p_ref[...][None]).astype(jnp.bfloat16)
    parts = [
        lax.slice(x, (0, 0, k * _D), (_BB, 2 * _D, (k + 1) * _D))
        for k in range(4)
    ]
    u = lax.broadcasted_iota(jnp.int32, (2 * _D, 256), 0)
    tt = lax.broadcasted_iota(jnp.int32, (2 * _D, 256), 1)
    acc = None
    for k in range(4):
        w = (u * 4 + k == tt).astype(jnp.bfloat16)
        y = lax.dot_general(
            parts[k], w, (((1,), (0,)), ((), ())),
            preferred_element_type=jnp.float32,
        )
        acc = y if acc is None else acc + y
    o_ref[...] = acc


_P = 4


def kernel(idx, token_table, pos_table):
    B, T = idx.shape
    N = B * T
    pblk = B // _P // _BB
    idx_flat = (
        idx.reshape(B // 8, 8, T // 128, 128).transpose(0, 2, 1, 3).reshape(-1)
    )
    pos64 = lax.slice(token_table, (0, 0), (T, _D)).reshape(2 * _D, 128)

    rows64 = [
        _sc_gather(
            idx_flat, token_table, N, p * N // _P, (p + 1) * N // _P
        ).reshape(B, 2 * _D, 128)
        for p in range(_P)
    ]

    out = None
    for p in range(_P):
        def in_map(i, p=p):
            return (i + p * pblk, 0, 0)

        in_specs = [
            pl.BlockSpec((_BB, 2 * _D, 128), in_map),
            pl.BlockSpec((2 * _D, 128), lambda i: (0, 0)),
        ]
        operands = [rows64[p], pos64]
        aliases = {}
        if out is not None:
            in_specs.append(pl.BlockSpec((1, _D, T), lambda i: (0, 0, 0)))
            operands.append(out)
            aliases = {2: 0}
        out = pl.pallas_call(
            _tc_body,
            grid=(pblk,),
            in_specs=in_specs,
            out_specs=pl.BlockSpec((_BB, _D, T), in_map),
            out_shape=jax.ShapeDtypeStruct((B, _D, T), jnp.float32),
            input_output_aliases=aliases,
        )(*operands)
    return out.transpose(0, 2, 1)


# speedup vs baseline: 1.0479x; 1.0067x over previous
import functools

import jax
import jax.numpy as jnp
from jax import lax
from jax.experimental import pallas as pl
from jax.experimental.pallas import tpu as pltpu
from jax.experimental.pallas import tpu_sc as plsc

_D = 32
_C = 1024
_BB = 128


def _sc_gather(idx_flat, token_table, N, lo, hi):
    NC, NS = 2, 16
    NW = NC * NS
    per_w = (hi - lo) // NW
    n_chunks = per_w // _C

    mesh = plsc.VectorSubcoreMesh(core_axis_name="c", subcore_axis_name="s")

    @functools.partial(
        pl.kernel,
        mesh=mesh,
        out_type=jax.ShapeDtypeStruct((N, _D), jnp.float32),
        scratch_types=[
            pltpu.VMEM((2, _C), jnp.int32),
            pltpu.VMEM((2, _C, _D), jnp.float32),
            pltpu.SemaphoreType.DMA,
            pltpu.SemaphoreType.DMA,
            pltpu.SemaphoreType.DMA,
            pltpu.SemaphoreType.DMA,
        ],
        compiler_params=pltpu.CompilerParams(use_tc_tiling_on_sc=False),
    )
    def k(idx_hbm, table_hbm, out_hbm, idx_v, rows_v, gsem0, gsem1, osem0, osem1):
        wid = lax.axis_index("s") * NC + lax.axis_index("c")
        base = lo + wid * per_w
        gsems = (gsem0, gsem1)
        osems = (osem0, osem1)

        def store_chunk(s, off, wait):
            gc = off // _C
            o0 = (gc // 2) * (8 * 256) + (gc % 2) * 128
            for r8 in range(8):
                args = (
                    rows_v.at[s].at[pl.ds(r8 * 128, 128)],
                    out_hbm.at[pl.ds(o0 + r8 * 256, 128)],
                    osems[s],
                )
                if wait:
                    pltpu.make_async_copy(*args).wait()
                else:
                    pltpu.async_copy(*args)

        pltpu.sync_copy(idx_hbm.at[pl.ds(base, _C)], idx_v.at[0])
        pltpu.async_copy(table_hbm.at[idx_v.at[0]], rows_v.at[0], gsem0)

        def chunk_pair(g2, carry):
            for s in (0, 1):
                g = 2 * g2 + s
                ns = 1 - s
                off = base + g * _C

                @pl.when((g + 1 < n_chunks) & (g >= 1))
                def _():
                    store_chunk(ns, off - _C, wait=True)

                @pl.when(g + 1 < n_chunks)
                def _():
                    pltpu.sync_copy(
                        idx_hbm.at[pl.ds(off + _C, _C)], idx_v.at[ns]
                    )
                    pltpu.async_copy(
                        table_hbm.at[idx_v.at[ns]], rows_v.at[ns], gsems[ns]
                    )

                pltpu.make_async_copy(
                    table_hbm.at[idx_v.at[s]], rows_v.at[s], gsems[s]
                ).wait()
                store_chunk(s, off, wait=False)
            return carry

        lax.fori_loop(0, n_chunks // 2, chunk_pair, 0)
        for g in (n_chunks - 2, n_chunks - 1):
            store_chunk(g % 2, base + g * _C, wait=True)

    return k(idx_flat, token_table)


def _tc_body(x_ref, p_ref, *rest):
    o_ref = rest[-1]
    x = (x_ref[...] + p_ref[...][None]).astype(jnp.bfloat16)
    parts = [
        lax.slice(x, (0, 0, k * _D), (_BB, 2 * _D, (k + 1) * _D))
        for k in range(4)
    ]
    u = lax.broadcasted_iota(jnp.int32, (2 * _D, 256), 0)
    tt = lax.broadcasted_iota(jnp.int32, (2 * _D, 256), 1)
    acc = None
    for k in range(4):
        w = (u * 4 + k == tt).astype(jnp.bfloat16)
        y = lax.dot_general(
            parts[k], w, (((1,), (0,)), ((), ())),
            preferred_element_type=jnp.float32,
        )
        acc = y if acc is None else acc + y
    o_ref[...] = acc


_P = 4


def kernel(idx, token_table, pos_table):
    B, T = idx.shape
    N = B * T
    pblk = B // _P // _BB
    idx_flat = (
        idx.reshape(B // 8, 8, T // 128, 128).transpose(0, 2, 1, 3).reshape(-1)
    )
    pos64 = lax.slice(token_table, (0, 0), (T, _D)).reshape(2 * _D, 128)

    rows64 = [
        _sc_gather(
            idx_flat, token_table, N, p * N // _P, (p + 1) * N // _P
        ).reshape(B, 2 * _D, 128)
        for p in range(_P)
    ]

    out = None
    for p in range(_P):
        def in_map(i, p=p):
            return (i + p * pblk, 0, 0)

        in_specs = [
            pl.BlockSpec((_BB, 2 * _D, 128), in_map),
            pl.BlockSpec((2 * _D, 128), lambda i: (0, 0)),
        ]
        operands = [rows64[p], pos64]
        aliases = {}
        if out is not None:
            in_specs.append(pl.BlockSpec((1, _D, T), lambda i: (0, 0, 0)))
            operands.append(out)
            aliases = {2: 0}
        out = pl.pallas_call(
            _tc_body,
            grid=(pblk,),
            in_specs=in_specs,
            out_specs=pl.BlockSpec((_BB, _D, T), in_map),
            out_shape=jax.ShapeDtypeStruct((B, _D, T), jnp.float32),
            input_output_aliases=aliases,
        )(*operands)
    return out.transpose(0, 2, 1)
